# baseline (speedup 1.0000x reference)
import functools

import jax
import jax.numpy as jnp
from jax import lax
from jax.experimental import pallas as pl
from jax.experimental.pallas import tpu as pltpu
from jax.experimental.pallas import tpu_sc as plsc

_SEGS = 64
_N = 50000
_D = 1024
_NW = 32
_CHUNK = 1568
_NPAD = _NW * _CHUNK

_BLOCK = 2048
_NBLK = -(-_N // _BLOCK)
_PSLOTS = 80


def _proj_body(x_ref, w_ref, h0_ref, h1_ref):
    hT = lax.dot_general(
        w_ref[...], x_ref[...], (((1,), (1,)), ((), ())),
        preferred_element_type=jnp.float32)
    h0_ref[...] = hT[0]
    h1_ref[...] = hT[1]


def _project(x, W):
    return pl.pallas_call(
        _proj_body,
        grid=(_NBLK,),
        in_specs=[
            pl.BlockSpec((_BLOCK, _D), lambda i: (i, 0)),
            pl.BlockSpec((2, _D), lambda i: (0, 0)),
        ],
        out_specs=[
            pl.BlockSpec((_BLOCK,), lambda i: (i,)),
            pl.BlockSpec((_BLOCK,), lambda i: (i,)),
        ],
        out_shape=[
            jax.ShapeDtypeStruct((_NPAD,), jnp.float32),
            jax.ShapeDtypeStruct((_NPAD,), jnp.float32),
        ],
        compiler_params=pltpu.CompilerParams(
            dimension_semantics=("arbitrary",)),
    )(x, W)


_sc_mesh = plsc.VectorSubcoreMesh(core_axis_name="c", subcore_axis_name="s")


@functools.partial(
    pl.kernel,
    out_type=jax.ShapeDtypeStruct((3 * _NW * _PSLOTS,), jnp.float32),
    mesh=_sc_mesh,
    compiler_params=pltpu.CompilerParams(needs_layout_passes=False),
    scratch_types=[
        pltpu.VMEM((_CHUNK,), jnp.float32),
        pltpu.VMEM((_CHUNK,), jnp.float32),
        pltpu.VMEM((_CHUNK,), jnp.int32),
        pltpu.VMEM((_PSLOTS,), jnp.float32),
        pltpu.VMEM((_PSLOTS,), jnp.float32),
        pltpu.VMEM((_PSLOTS,), jnp.float32),
    ],
)
def _seg_partials(h0_hbm, h1_hbm, batch_hbm, out_hbm, h0_v, h1_v, idx_v,
                  a0_v, a1_v, ac_v):
    wid = lax.axis_index("c") * 16 + lax.axis_index("s")
    base = wid * _CHUNK
    pltpu.sync_copy(h0_hbm.at[pl.ds(base, _CHUNK)], h0_v)
    pltpu.sync_copy(h1_hbm.at[pl.ds(base, _CHUNK)], h1_v)
    pltpu.sync_copy(batch_hbm.at[pl.ds(base, _CHUNK)], idx_v)

    zeros16 = jnp.zeros((16,), jnp.float32)
    for c in range(_PSLOTS // 16):
        a0_v[pl.ds(c * 16, 16)] = zeros16
        a1_v[pl.ds(c * 16, 16)] = zeros16
        ac_v[pl.ds(c * 16, 16)] = zeros16

    ones16 = jnp.ones((16,), jnp.float32)

    def body(j, carry):
        off = j * 16
        vidx = idx_v[pl.ds(off, 16)]
        plsc.addupdate_scatter(a0_v, [vidx], h0_v[pl.ds(off, 16)])
        plsc.addupdate_scatter(a1_v, [vidx], h1_v[pl.ds(off, 16)])
        plsc.addupdate_scatter(ac_v, [vidx], ones16)
        return carry

    lax.fori_loop(0, _CHUNK // 16, body, 0, unroll=4)

    pltpu.sync_copy(a0_v, out_hbm.at[pl.ds(wid * _PSLOTS, _PSLOTS)])
    pltpu.sync_copy(a1_v, out_hbm.at[pl.ds(_NW * _PSLOTS + wid * _PSLOTS,
                                           _PSLOTS)])
    pltpu.sync_copy(ac_v, out_hbm.at[pl.ds(2 * _NW * _PSLOTS + wid * _PSLOTS,
                                           _PSLOTS)])


def _combine_body(part_ref, b_ref, out_ref):
    s = jnp.sum(part_ref[...], axis=1)
    sums = s[0:2, :]
    cnt = s[2:3, :]
    out_ref[...] = (sums + cnt * b_ref[...]) / jnp.maximum(cnt, 1.0)


def _combine(partials, b):
    return pl.pallas_call(
        _combine_body,
        in_specs=[
            pl.BlockSpec((3, _NW, _PSLOTS), lambda: (0, 0, 0)),
            pl.BlockSpec((2, 1), lambda: (0, 0)),
        ],
        out_specs=pl.BlockSpec((2, _PSLOTS), lambda: (0, 0)),
        out_shape=jax.ShapeDtypeStruct((2, _PSLOTS), jnp.float32),
    )(partials, b)


def kernel(x, edge_index, batch, W, b):
    h0, h1 = _project(x, W)
    batch_pad = jnp.concatenate(
        [batch, jnp.full((_NPAD - _N,), _SEGS, jnp.int32)])
    partials = _seg_partials(h0, h1, batch_pad)
    out2 = _combine(partials.reshape(3, _NW, _PSLOTS), b.reshape(2, 1))
    return out2[:, :_SEGS].T


# speedup vs baseline: 1.0533x; 1.0533x over previous
import functools

import jax
import jax.numpy as jnp
from jax import lax
from jax.experimental import pallas as pl
from jax.experimental.pallas import tpu as pltpu
from jax.experimental.pallas import tpu_sc as plsc

_SEGS = 64
_N = 50000
_D = 1024

_NW = 32
_RB = 16
_KS = 98
_NPAD = _NW * _RB * _KS
_PSLOTS = 80

_BT = 2000
_NTB = _N // _BT


_sc_mesh = plsc.VectorSubcoreMesh(core_axis_name="c", subcore_axis_name="s")


@functools.partial(
    pl.kernel,
    out_type=jax.ShapeDtypeStruct((_NW * _PSLOTS,), jnp.float32),
    mesh=_sc_mesh,
    compiler_params=pltpu.CompilerParams(needs_layout_passes=False),
    scratch_types=[
        pltpu.VMEM((_KS, _RB), jnp.int32),
        pltpu.VMEM((_PSLOTS,), jnp.float32),
    ],
)
def _sc_counts(batch3_hbm, cnt_hbm, bv_v, cnt_v):
    cid = lax.axis_index("c")
    sid = lax.axis_index("s")
    wid = cid * 16 + sid

    pltpu.sync_copy(batch3_hbm.at[wid], bv_v)

    zeros16 = jnp.zeros((16,), jnp.float32)
    for c in range(_PSLOTS // 16):
        cnt_v[pl.ds(c * 16, 16)] = zeros16

    ones16 = jnp.ones((16,), jnp.float32)

    def body(k, carry):
        vidx = bv_v[k, pl.ds(0, _RB)]
        plsc.addupdate_scatter(cnt_v, [vidx], ones16)
        return carry

    lax.fori_loop(0, _KS, body, 0, unroll=4)

    pltpu.sync_copy(cnt_v, cnt_hbm.at[pl.ds(wid * _PSLOTS, _PSLOTS)])


def _tc_body(x_ref, batch_ref, w_ref, sums_ref):
    i = pl.program_id(0)

    @pl.when(i == 0)
    def _():
        sums_ref[...] = jnp.zeros_like(sums_ref)

    h2 = lax.dot_general(w_ref[...], x_ref[...], (((1,), (1,)), ((), ())),
                         preferred_element_type=jnp.float32)
    bidx = batch_ref[0]
    seg = lax.broadcasted_iota(jnp.int32, (_SEGS, _BT), 0)
    onehot = (bidx == seg).astype(jnp.float32)
    psum = lax.dot_general(h2, onehot, (((1,), (1,)), ((), ())),
                           preferred_element_type=jnp.float32)
    sums_ref[...] += psum


def _tc_sums(x, batch3, W):
    return pl.pallas_call(
        _tc_body,
        grid=(_NTB,),
        in_specs=[
            pl.BlockSpec((_BT, _D), lambda i: (i, 0)),
            pl.BlockSpec((1, 1, _BT), lambda i: (i, 0, 0)),
            pl.BlockSpec((2, _D), lambda i: (0, 0)),
        ],
        out_specs=pl.BlockSpec((2, _SEGS), lambda i: (0, 0)),
        out_shape=jax.ShapeDtypeStruct((2, _SEGS), jnp.float32),
        compiler_params=pltpu.CompilerParams(
            dimension_semantics=("arbitrary",)),
    )(x, batch3, W)


def _combine_body(cntsc_ref, tcs_ref, b_ref, out_ref):
    csc = jnp.sum(cntsc_ref[...], axis=0)
    cnt = csc[:_SEGS][None, :]
    out_ref[...] = (tcs_ref[...] + cnt * b_ref[...]) / jnp.maximum(cnt, 1.0)


def _combine(cnt_sc2, tc_sums, b2):
    return pl.pallas_call(
        _combine_body,
        in_specs=[
            pl.BlockSpec((_NW, _PSLOTS), lambda: (0, 0)),
            pl.BlockSpec((2, _SEGS), lambda: (0, 0)),
            pl.BlockSpec((2, 1), lambda: (0, 0)),
        ],
        out_specs=pl.BlockSpec((2, _SEGS), lambda: (0, 0)),
        out_shape=jax.ShapeDtypeStruct((2, _SEGS), jnp.float32),
    )(cnt_sc2, tc_sums, b2)


def kernel(x, edge_index, batch, W, b):
    batch_pad = jnp.concatenate(
        [batch, jnp.full((_NPAD - _N,), _SEGS, jnp.int32)])
    cnt_sc = _sc_counts(batch_pad.reshape(_NW, _KS, _RB))
    tc_sums = _tc_sums(x, batch.reshape(_NTB, 1, _BT), W)
    out2 = _combine(cnt_sc.reshape(_NW, _PSLOTS), tc_sums, b.reshape(2, 1))
    return out2.T
